# Initial kernel scaffold; baseline (speedup 1.0000x reference)
import jax
import jax.numpy as jnp
from jax.experimental import pallas as pl


def kernel(duration, returns, direction, start_at, stop_at, batch_idx, market):
    raise NotImplementedError("write your pallas kernel here")


# speedup vs baseline: 13.2938x; 13.2938x over previous
import functools

import jax
import jax.numpy as jnp
from jax import lax
from jax.experimental import pallas as pl
from jax.experimental.pallas import tpu as pltpu
from jax.experimental.pallas import tpu_sc as plsc

T, B, M = 2048, 32, 32
BM = B * M
LOW_DURATION, HIGH_DURATION = 10, 30
HIGH_RETURNS, LOW_RETURNS = 0.01, 0.005

N_CORES = 2
N_SUB = 16
T_HALF = T // N_CORES
CHUNKS = ((0, 512), (512, 512))
PLANE_WORDS = 512 * BM + 64
ZERO_N = 1024
EBLK = 272


def _sc_body(dur_h, ret_h, dir_h, sta_h, sto_h, bi_h, mk_h,
             o_up, o_sd, o_dn, o_mw,
             dur_v, ret_v, dir_v, sta_v, sto_v, bi_v, mk_v,
             fidx_v, vu_v, vs_v, vd_v, vm_v, idx_v, zero_v, acc):
    E = fidx_v.shape[0] // 2
    P = 2 * E
    c = lax.axis_index("c")
    s = lax.axis_index("s")
    base = s * E

    def _zfill(i, carry):
        zero_v[pl.ds(i * 16, 16)] = jnp.zeros((16,), jnp.float32)
        return carry
    lax.fori_loop(0, ZERO_N // 16, _zfill, 0)

    def _build(g, carry):
        blk = g // (EBLK // 16)
        gg = g % (EBLK // 16)

        @pl.when(gg == 0)
        def _stage():
            eb = base + blk * EBLK
            pltpu.sync_copy(dur_h.at[pl.ds(eb, EBLK)], dur_v)
            pltpu.sync_copy(ret_h.at[pl.ds(eb, EBLK)], ret_v)
            pltpu.sync_copy(dir_h.at[pl.ds(eb, EBLK)], dir_v)
            pltpu.sync_copy(sta_h.at[pl.ds(eb, EBLK)], sta_v)
            pltpu.sync_copy(sto_h.at[pl.ds(eb, EBLK)], sto_v)
            pltpu.sync_copy(bi_h.at[pl.ds(eb, EBLK)], bi_v)
            pltpu.sync_copy(mk_h.at[pl.ds(eb, EBLK)], mk_v)

        o = g * 16
        ob = gg * 16
        du = dur_v[pl.ds(ob, 16)]
        re = ret_v[pl.ds(ob, 16)]
        di = dir_v[pl.ds(ob, 16)]
        st = sta_v[pl.ds(ob, 16)]
        sp = sto_v[pl.ds(ob, 16)]
        bm = bi_v[pl.ds(ob, 16)] * M + mk_v[pl.ds(ob, 16)]

        one = jnp.float32(1.0)
        dsc = jnp.where(du > HIGH_DURATION, one,
                        jnp.where(du < LOW_DURATION, jnp.float32(0.5),
                                  jnp.float32(0.75)))
        rsc = jnp.where(re > HIGH_RETURNS, one,
                        jnp.where(re < LOW_RETURNS, jnp.float32(0.0),
                                  jnp.float32(0.75)))
        ts = dsc * rsc
        lowret = jnp.where(re < LOW_RETURNS, one, jnp.float32(0.0))
        highdur = jnp.where(du > HIGH_DURATION, one, jnp.float32(0.0))
        midret = jnp.where(re < HIGH_RETURNS, one, jnp.float32(0.0))
        fs = lowret + (one - lowret) * highdur * midret
        nside = one - fs
        fu = nside * jnp.where(di > 0, one, jnp.float32(0.0))
        fd = nside * jnp.where(di < 0, one, jnp.float32(0.0))
        fm = ts * (fu + fs + fd)

        fidx_v[pl.ds(o, 16)] = st * BM + bm
        fidx_v[pl.ds(E + o, 16)] = sp * BM + bm
        vu_v[pl.ds(o, 16)] = fu
        vu_v[pl.ds(E + o, 16)] = -fu
        vs_v[pl.ds(o, 16)] = fs
        vs_v[pl.ds(E + o, 16)] = -fs
        vd_v[pl.ds(o, 16)] = fd
        vd_v[pl.ds(E + o, 16)] = -fd
        vm_v[pl.ds(o, 16)] = fm
        vm_v[pl.ds(E + o, 16)] = -fm
        return carry
    lax.fori_loop(0, E // 16, _build, 0)

    lane = lax.iota(jnp.int32, 16)
    half_row0 = c * T_HALF
    vals = (vu_v, vs_v, vd_v, vm_v)
    outs = (o_up, o_sd, o_dn, o_mw)

    for cb, tc in CHUNKS:
        span = tc * BM // N_SUB
        lo = (half_row0 + cb) * BM
        hi = lo + tc * BM
        dummy_base = tc * BM

        def _mkidx(g, carry):
            o = g * 16
            f = fidx_v[pl.ds(o, 16)]
            dmy = dummy_base + lane + (g % 4) * 16
            idx_v[pl.ds(o, 16)] = jnp.where(
                f >= lo, jnp.where(f < hi, f - lo, dmy), dmy)
            return carry
        lax.fori_loop(0, P // 16, _mkidx, 0)

        for v_v, o_h in zip(vals, outs):
            for j in range(span // ZERO_N):
                pltpu.sync_copy(
                    zero_v, acc.at[pl.ds(s * span + j * ZERO_N, ZERO_N)])
            plsc.subcore_barrier()
            pltpu.sync_copy(v_v, acc.at[idx_v], add=True)
            plsc.subcore_barrier()
            pltpu.sync_copy(acc.at[pl.ds(s * span, span)],
                            o_h.at[pl.ds(lo + s * span, span)])
            plsc.subcore_barrier()


def _sc_scatter(dur, ret, dirn, sta, sto, bi, mk):
    np_ = dur.shape[0]
    e = np_ // N_SUB
    mesh = plsc.VectorSubcoreMesh(core_axis_name="c", subcore_axis_name="s")
    f32, i32 = jnp.float32, jnp.int32
    out = jax.ShapeDtypeStruct((T * BM,), f32)
    kern = functools.partial(
        pl.kernel,
        out_type=[out, out, out, out],
        mesh=mesh,
        scratch_types=[
            pltpu.VMEM((EBLK,), i32),
            pltpu.VMEM((EBLK,), f32),
            pltpu.VMEM((EBLK,), f32),
            pltpu.VMEM((EBLK,), i32),
            pltpu.VMEM((EBLK,), i32),
            pltpu.VMEM((EBLK,), i32),
            pltpu.VMEM((EBLK,), i32),
            pltpu.VMEM((2 * e,), i32),
            pltpu.VMEM((2 * e,), f32),
            pltpu.VMEM((2 * e,), f32),
            pltpu.VMEM((2 * e,), f32),
            pltpu.VMEM((2 * e,), f32),
            pltpu.VMEM((2 * e,), i32),
            pltpu.VMEM((ZERO_N,), f32),
            pltpu.VMEM_SHARED((PLANE_WORDS,), f32),
        ],
    )(_sc_body)
    return kern(dur, ret, dirn, sta, sto, bi, mk)


def _tc_cumsum_body(u_ref, s_ref, d_ref, m_ref,
                    ou_ref, os_ref, od_ref, om_ref, carry_ref):
    tb = pl.program_id(0)

    @pl.when(tb == 0)
    def _():
        carry_ref[...] = jnp.zeros_like(carry_ref)

    n = u_ref.shape[0]
    r = lax.broadcasted_iota(jnp.int32, (n, n), 0)
    q = lax.broadcasted_iota(jnp.int32, (n, n), 1)
    ltri = jnp.where(r >= q, jnp.float32(1.0), jnp.float32(0.0))
    ins = (u_ref, s_ref, d_ref, m_ref)
    outs = (ou_ref, os_ref, od_ref, om_ref)
    for i in range(4):
        x = ins[i][...]
        cs = jnp.dot(ltri, x, preferred_element_type=jnp.float32)
        cs = cs + carry_ref[i:i + 1, :]
        outs[i][...] = cs
        carry_ref[i:i + 1, :] = cs[n - 1:n, :]


def _tc_cumsum(u, sd, dn, mw):
    tb = 256
    spec = pl.BlockSpec((tb, BM), lambda t: (t, 0))
    out = jax.ShapeDtypeStruct((T, BM), jnp.float32)
    return pl.pallas_call(
        _tc_cumsum_body,
        grid=(T // tb,),
        in_specs=[spec, spec, spec, spec],
        out_specs=[spec, spec, spec, spec],
        out_shape=[out, out, out, out],
        scratch_shapes=[pltpu.VMEM((4, BM), jnp.float32)],
    )(u, sd, dn, mw)


def kernel(duration, returns, direction, start_at, stop_at, batch_idx, market):
    n = duration.shape[0]
    grain = N_SUB * EBLK
    np_ = ((n + grain - 1) // grain) * grain
    pad = np_ - n

    def pz(x, dt):
        x = x.astype(dt)
        return jnp.pad(x, (0, pad)) if pad else x

    dur = pz(duration, jnp.int32)
    ret = pz(returns, jnp.float32)
    dirn = pz(direction, jnp.float32)
    sta = pz(start_at, jnp.int32)
    sto = pz(stop_at, jnp.int32)
    bi = pz(batch_idx, jnp.int32)
    mk = pz(market, jnp.int32)

    pu, ps, pd, pm = _sc_scatter(dur, ret, dirn, sta, sto, bi, mk)
    cu, cs, cd, cm = _tc_cumsum(pu.reshape(T, BM), ps.reshape(T, BM),
                                pd.reshape(T, BM), pm.reshape(T, BM))
    labels = jnp.stack([cu, cs, cd], axis=-1).reshape(T, B, M, 3)
    mask_cum = cm.reshape(T, B, M)
    return labels, mask_cum
